# baseline (speedup 1.0000x reference)
import functools

import jax
import jax.numpy as jnp
from jax import lax
from jax.experimental import pallas as pl
from jax.experimental.pallas import tpu as pltpu
from jax.experimental.pallas import tpu_sc as plsc

NC = 2
NS = 16
NW = NC * NS
LANES = 16

VOCAB = 28436
VOCAB_DIM = 300
PIECE = 128
NPIECE = 3
TAIL = VOCAB_DIM - 2 * PIECE
VB = 512
NVB = (VOCAB + VB - 1) // VB
BATCH = 16384
PER_W = BATCH // NW
CHUNK = 64
NCHUNK = PER_W // CHUNK
NBUF = 2


def _tp_full_body(in_ref, out_ref):
    out_ref[...] = jnp.swapaxes(in_ref[...], 0, 1)


def _tp_tail_body(in_ref, out_ref):
    x = in_ref[...]
    y = jnp.swapaxes(x[:TAIL], 0, 1)
    z = jnp.zeros((VB, PIECE - TAIL), jnp.float32)
    out_ref[...] = jnp.concatenate([y, z], axis=1)


@functools.partial(jax.jit, static_argnums=1)
def _transpose_piece(tt, k):
    body = _tp_tail_body if k == 2 else _tp_full_body
    return pl.pallas_call(
        body,
        grid=(NVB,),
        in_specs=[pl.BlockSpec((PIECE, VB), lambda i, k=k: (k, i))],
        out_specs=pl.BlockSpec((VB, PIECE), lambda i: (i, 0)),
        out_shape=jax.ShapeDtypeStruct((VOCAB, PIECE), jnp.float32),
    )(tt)


def _sc_body(x0_hbm, x1_hbm, tbl_hbm, out_hbm,
             idx0_v, idx1_v, a0_v, b0_v, a1_v, b1_v,
             stage_v, s0, s1, sg):
    wid = lax.axis_index("s") * NC + lax.axis_index("c")
    base = wid * PER_W
    ca = pltpu.async_copy(x0_hbm.at[pl.ds(base, PER_W)], idx0_v, s0)
    cb = pltpu.async_copy(x1_hbm.at[pl.ds(base, PER_W)], idx1_v, s1)
    ca.wait()
    cb.wait()

    bufs = [(a0_v, b0_v), (a1_v, b1_v)]
    sems = [s0, s1]

    def issue(g):
        slot = g % NBUF
        i0 = idx0_v.at[pl.ds(g * CHUNK, CHUNK)]
        i1 = idx1_v.at[pl.ds(g * CHUNK, CHUNK)]
        av, bv = bufs[slot]
        sem = sems[slot]
        return (
            pltpu.async_copy(tbl_hbm.at[i0], av, sem),
            pltpu.async_copy(tbl_hbm.at[i1], bv, sem),
        )

    acc = jnp.zeros((LANES,), jnp.float32)
    pending = {}
    for g in range(min(NBUF, NCHUNK)):
        pending[g] = issue(g)
    for g in range(NCHUNK):
        slot = g % NBUF
        for h in pending.pop(g):
            h.wait()
        av, bv = bufs[slot]

        def row_step(r, acc, av=av, bv=bv):
            for j in range(PIECE // LANES):
                a = av[r, pl.ds(j * LANES, LANES)]
                b = bv[r, pl.ds(j * LANES, LANES)]
                acc = acc + a * b
            return acc

        acc = lax.fori_loop(0, CHUNK, row_step, acc)
        if g + NBUF < NCHUNK:
            pending[g + NBUF] = issue(g + NBUF)

    for i in range(8):
        for j in range(128 // LANES):
            stage_v[i, pl.ds(j * LANES, LANES)] = jnp.zeros(
                (LANES,), jnp.float32)
    stage_v[0, pl.ds(0, LANES)] = acc
    pltpu.async_copy(stage_v, out_hbm.at[wid], sg).wait()


@jax.jit
def _sc_gather_dot(x0, x1, piece):
    mesh = plsc.VectorSubcoreMesh(core_axis_name="c", subcore_axis_name="s")
    return pl.kernel(
        _sc_body,
        out_type=jax.ShapeDtypeStruct((NW, 8, 128), jnp.float32),
        mesh=mesh,
        scratch_types=[
            pltpu.VMEM((PER_W,), jnp.int32),
            pltpu.VMEM((PER_W,), jnp.int32),
            pltpu.VMEM((CHUNK, PIECE), jnp.float32),
            pltpu.VMEM((CHUNK, PIECE), jnp.float32),
            pltpu.VMEM((CHUNK, PIECE), jnp.float32),
            pltpu.VMEM((CHUNK, PIECE), jnp.float32),
            pltpu.VMEM((8, 128), jnp.float32),
            pltpu.SemaphoreType.DMA,
            pltpu.SemaphoreType.DMA,
            pltpu.SemaphoreType.DMA,
        ],
    )(x0, x1, piece)


def _finish_body(p0_ref, p1_ref, p2_ref, o_ref):
    s = jnp.sum(p0_ref[...]) + jnp.sum(p1_ref[...]) + jnp.sum(p2_ref[...])
    o_ref[0, 0] = s * s


@jax.jit
def _finish(p0, p1, p2):
    out = pl.pallas_call(
        _finish_body,
        out_shape=jax.ShapeDtypeStruct((1, 1), jnp.float32),
        out_specs=pl.BlockSpec(memory_space=pltpu.SMEM),
    )(p0, p1, p2)
    return out[0, 0]


def kernel(x, table):
    x0 = x[:, 0]
    x1 = x[:, 1]
    tt = table.T
    partials = [
        _sc_gather_dot(x0, x1, _transpose_piece(tt, k))
        for k in range(NPIECE)
    ]
    return _finish(*partials)


# speedup vs baseline: 1.3944x; 1.3944x over previous
import functools

import jax
import jax.numpy as jnp
from jax import lax
from jax.experimental import pallas as pl
from jax.experimental.pallas import tpu as pltpu
from jax.experimental.pallas import tpu_sc as plsc

NC = 2
NS = 16
NW = NC * NS
LANES = 16

VOCAB_DIM = 300
MAIN = 256
TW = 128
NTS = 3
BATCH = 16384
PER_W = BATCH // NW
CHUNK = 64
NCHUNK = PER_W // CHUNK
NBUF = 2


def _sc_body(x0_hbm, x1_hbm, tbl_hbm, ttl_hbm, out_hbm,
             idx0_v, idx1_v,
             ma0_v, mb0_v, ta0_v, tb0_v,
             ma1_v, mb1_v, ta1_v, tb1_v,
             stage_v, s0, s1, sg):
    wid = lax.axis_index("s") * NC + lax.axis_index("c")
    base = wid * PER_W
    ca = pltpu.async_copy(x0_hbm.at[pl.ds(base, PER_W)], idx0_v, s0)
    cb = pltpu.async_copy(x1_hbm.at[pl.ds(base, PER_W)], idx1_v, s1)
    ca.wait()
    cb.wait()

    bufs = [(ma0_v, mb0_v, ta0_v, tb0_v), (ma1_v, mb1_v, ta1_v, tb1_v)]
    sems = [s0, s1]

    def issue(g):
        slot = g % NBUF
        i0 = idx0_v.at[pl.ds(g * CHUNK, CHUNK)]
        i1 = idx1_v.at[pl.ds(g * CHUNK, CHUNK)]
        ma, mb, ta, tb = bufs[slot]
        sem = sems[slot]
        return (
            pltpu.async_copy(tbl_hbm.at[i0, pl.ds(0, MAIN)], ma, sem),
            pltpu.async_copy(tbl_hbm.at[i1, pl.ds(0, MAIN)], mb, sem),
            pltpu.async_copy(ttl_hbm.at[i0], ta, sem),
            pltpu.async_copy(ttl_hbm.at[i1], tb, sem),
        )

    acc = jnp.zeros((LANES,), jnp.float32)
    pending = {}
    for g in range(min(NBUF, NCHUNK)):
        pending[g] = issue(g)
    for g in range(NCHUNK):
        slot = g % NBUF
        for h in pending.pop(g):
            h.wait()
        ma, mb, ta, tb = bufs[slot]

        def row_step(r, acc, ma=ma, mb=mb, ta=ta, tb=tb):
            for j in range(MAIN // LANES):
                a = ma[r, pl.ds(j * LANES, LANES)]
                b = mb[r, pl.ds(j * LANES, LANES)]
                acc = acc + a * b
            for j in range(NTS):
                a = ta[r, pl.ds(j * LANES, LANES)]
                b = tb[r, pl.ds(j * LANES, LANES)]
                acc = acc + a * b
            return acc

        acc = lax.fori_loop(0, CHUNK, row_step, acc)
        if g + NBUF < NCHUNK:
            pending[g + NBUF] = issue(g + NBUF)

    for i in range(8):
        for j in range(128 // LANES):
            stage_v[i, pl.ds(j * LANES, LANES)] = jnp.zeros(
                (LANES,), jnp.float32)
    stage_v[0, pl.ds(0, LANES)] = acc
    pltpu.async_copy(stage_v, out_hbm.at[wid], sg).wait()


@jax.jit
def _sc_gather_dot(x0, x1, table, tail_tbl):
    mesh = plsc.VectorSubcoreMesh(core_axis_name="c", subcore_axis_name="s")
    return pl.kernel(
        _sc_body,
        out_type=jax.ShapeDtypeStruct((NW, 8, 128), jnp.float32),
        mesh=mesh,
        scratch_types=[
            pltpu.VMEM((PER_W,), jnp.int32),
            pltpu.VMEM((PER_W,), jnp.int32),
            pltpu.VMEM((CHUNK, MAIN), jnp.float32),
            pltpu.VMEM((CHUNK, MAIN), jnp.float32),
            pltpu.VMEM((CHUNK, TW), jnp.float32),
            pltpu.VMEM((CHUNK, TW), jnp.float32),
            pltpu.VMEM((CHUNK, MAIN), jnp.float32),
            pltpu.VMEM((CHUNK, MAIN), jnp.float32),
            pltpu.VMEM((CHUNK, TW), jnp.float32),
            pltpu.VMEM((CHUNK, TW), jnp.float32),
            pltpu.VMEM((8, 128), jnp.float32),
            pltpu.SemaphoreType.DMA,
            pltpu.SemaphoreType.DMA,
            pltpu.SemaphoreType.DMA,
        ],
    )(x0, x1, table, tail_tbl)


def _finish_body(p_ref, o_ref):
    s = jnp.sum(p_ref[...])
    o_ref[0, 0] = s * s


@jax.jit
def _finish(partials):
    out = pl.pallas_call(
        _finish_body,
        out_shape=jax.ShapeDtypeStruct((1, 1), jnp.float32),
        out_specs=pl.BlockSpec(memory_space=pltpu.SMEM),
    )(partials)
    return out[0, 0]


def kernel(x, table):
    x0 = x[:, 0]
    x1 = x[:, 1]
    tail_tbl = jnp.pad(lax.optimization_barrier(table)[:, MAIN:],
                       ((0, 0), (0, TW - (VOCAB_DIM - MAIN))))
    partials = _sc_gather_dot(x0, x1, table, tail_tbl)
    return _finish(partials)


# speedup vs baseline: 1.5714x; 1.1270x over previous
import functools

import jax
import jax.numpy as jnp
from jax import lax
from jax.experimental import pallas as pl
from jax.experimental.pallas import tpu as pltpu
from jax.experimental.pallas import tpu_sc as plsc

NC = 2
NS = 16
NW = NC * NS
LANES = 16

VOCAB_DIM = 300
MAIN = 256
TW = 128
NTS = 3
BATCH = 16384
PER_W = BATCH // NW
CHUNK = 64
NCHUNK = PER_W // CHUNK
NBUF = 2


def _sc_body(x0_hbm, x1_hbm, tbl_hbm, ttl_hbm, out_hbm,
             idx0_v, idx1_v,
             ma0_v, mb0_v, ta0_v, tb0_v,
             ma1_v, mb1_v, ta1_v, tb1_v,
             stage_v, s0, s1, sg):
    wid = lax.axis_index("s") * NC + lax.axis_index("c")
    base = wid * PER_W
    ca = pltpu.async_copy(x0_hbm.at[pl.ds(base, PER_W)], idx0_v, s0)
    cb = pltpu.async_copy(x1_hbm.at[pl.ds(base, PER_W)], idx1_v, s1)
    ca.wait()
    cb.wait()

    bufs = [(ma0_v, mb0_v, ta0_v, tb0_v), (ma1_v, mb1_v, ta1_v, tb1_v)]
    sems = [s0, s1]

    def issue(g):
        slot = g % NBUF
        i0 = idx0_v.at[pl.ds(g * CHUNK, CHUNK)]
        i1 = idx1_v.at[pl.ds(g * CHUNK, CHUNK)]
        ma, mb, ta, tb = bufs[slot]
        sem = sems[slot]
        return (
            pltpu.async_copy(tbl_hbm.at[i0, pl.ds(0, MAIN)], ma, sem),
            pltpu.async_copy(tbl_hbm.at[i1, pl.ds(0, MAIN)], mb, sem),
            pltpu.async_copy(ttl_hbm.at[i0], ta, sem),
            pltpu.async_copy(ttl_hbm.at[i1], tb, sem),
        )

    acc = jnp.zeros((LANES,), jnp.float32)
    pending = {}
    for g in range(min(NBUF, NCHUNK)):
        pending[g] = issue(g)
    for g in range(NCHUNK):
        slot = g % NBUF
        for h in pending.pop(g):
            h.wait()
        ma, mb, ta, tb = bufs[slot]

        def row_step(r, acc, ma=ma, mb=mb, ta=ta, tb=tb):
            for j in range(MAIN // LANES):
                a = ma[r, pl.ds(j * LANES, LANES)]
                b = mb[r, pl.ds(j * LANES, LANES)]
                acc = acc + a * b
            for j in range(NTS):
                a = ta[r, pl.ds(j * LANES, LANES)]
                b = tb[r, pl.ds(j * LANES, LANES)]
                acc = acc + a * b
            return acc

        acc = lax.fori_loop(0, CHUNK, row_step, acc)
        if g + NBUF < NCHUNK:
            pending[g + NBUF] = issue(g + NBUF)

    for i in range(8):
        for j in range(128 // LANES):
            stage_v[i, pl.ds(j * LANES, LANES)] = jnp.zeros(
                (LANES,), jnp.float32)
    stage_v[0, pl.ds(0, LANES)] = acc
    pltpu.async_copy(stage_v, out_hbm.at[wid], sg).wait()


@jax.jit
def _sc_gather_dot(x0, x1, table, tail_tbl):
    mesh = plsc.VectorSubcoreMesh(core_axis_name="c", subcore_axis_name="s")
    return pl.kernel(
        _sc_body,
        out_type=jax.ShapeDtypeStruct((NW, 8, 128), jnp.float32),
        mesh=mesh,
        scratch_types=[
            pltpu.VMEM((PER_W,), jnp.int32),
            pltpu.VMEM((PER_W,), jnp.int32),
            pltpu.VMEM((CHUNK, MAIN), jnp.float32),
            pltpu.VMEM((CHUNK, MAIN), jnp.float32),
            pltpu.VMEM((CHUNK, TW), jnp.float32),
            pltpu.VMEM((CHUNK, TW), jnp.float32),
            pltpu.VMEM((CHUNK, MAIN), jnp.float32),
            pltpu.VMEM((CHUNK, MAIN), jnp.float32),
            pltpu.VMEM((CHUNK, TW), jnp.float32),
            pltpu.VMEM((CHUNK, TW), jnp.float32),
            pltpu.VMEM((8, 128), jnp.float32),
            pltpu.SemaphoreType.DMA,
            pltpu.SemaphoreType.DMA,
            pltpu.SemaphoreType.DMA,
        ],
    )(x0, x1, table, tail_tbl)


def _finish_body(p_ref, o_ref):
    s = jnp.sum(p_ref[...])
    o_ref[0, 0] = s * s


@jax.jit
def _finish(partials):
    out = pl.pallas_call(
        _finish_body,
        out_shape=jax.ShapeDtypeStruct((1, 1), jnp.float32),
        out_specs=pl.BlockSpec(memory_space=pltpu.SMEM),
    )(partials)
    return out[0, 0]


def kernel(x, table):
    x0 = x[:, 0]
    x1 = x[:, 1]
    tail_tbl = jnp.pad(table[:, MAIN:],
                       ((0, 0), (0, TW - (VOCAB_DIM - MAIN))))
    partials = _sc_gather_dot(x0, x1, table, tail_tbl)
    return _finish(partials)
